# baseline (speedup 1.0000x reference)
import math

import jax
import jax.numpy as jnp
from jax import lax
from jax.experimental import pallas as pl
from jax.experimental.pallas import tpu as pltpu
from jax.experimental.pallas import tpu_sc as plsc

VOCAB = 1000000
EMB = 64
BATCH = 4096
SEQ = 200
SCALE = math.sqrt(EMB)

NC = 2
NS = 16
NW = NC * NS
LANES = 16

TCOLS = (VOCAB + 127) // 128
FULL_COLS = VOCAB // 128
RM_ROWS = TCOLS * 64
TAIL_ROW0 = FULL_COLS * 64

G1 = 4
N1 = 62

_MESH = plsc.VectorSubcoreMesh(
    core_axis_name="c", subcore_axis_name="s", num_cores=NC, num_subcores=NS
)
_PARAMS = pltpu.CompilerParams(
    use_tc_tiling_on_sc=True, needs_layout_passes=False)


def _wid():
    return lax.axis_index("s") * NC + lax.axis_index("c")


def _splat(x):
    return jnp.zeros((LANES,), jnp.int32) + x


def _transpose_block(ibuf, obuf, row_id, n_q):
    def qbody(q, carry):
        for half in range(2):
            col = _splat(2 * q + half)
            for l in range(4):
                v = plsc.load_gather(ibuf, [row_id[l], col])
                obuf[q, pl.ds(half * 64 + l * 16, 16)] = v
        return carry

    lax.fori_loop(0, n_q, qbody, 0)


def _detr_body(tabT, tabRM, i0, i1, i2, i3, o0, o1, o2, o3, tib, tob,
               isem, osem, tsem):
    w = _wid()
    lanes = lax.iota(jnp.int32, LANES)
    row_id = [_splat(l * 16) + lanes for l in range(4)]
    ibufs = (i0, i1, i2, i3)
    obufs = (o0, o1, o2, o3)

    def body(k, carry):
        cols = [w + NW * (G1 * k + i) for i in range(G1)]
        for i in range(G1):
            c = cols[i]

            @pl.when(c < FULL_COLS)
            def _():
                pltpu.async_copy(
                    tabT.at[:, pl.ds(pl.multiple_of(c * 128, 128), 128)],
                    ibufs[i], isem)

        for i in range(G1):
            c = cols[i]
            cprev = c - NW * G1

            @pl.when(c < FULL_COLS)
            def _():
                pltpu.make_async_copy(
                    tabT.at[:, pl.ds(pl.multiple_of(c * 128, 128), 128)],
                    ibufs[i], isem).wait()

            @pl.when(cprev >= 0)
            def _():
                pltpu.make_async_copy(
                    obufs[i],
                    tabRM.at[pl.ds(pl.multiple_of(cprev * 64, 8), 64)],
                    osem).wait()

            @pl.when(c < FULL_COLS)
            def _():
                _transpose_block(ibufs[i], obufs[i], row_id, 64)
                pltpu.async_copy(
                    obufs[i],
                    tabRM.at[pl.ds(pl.multiple_of(c * 64, 8), 64)],
                    osem)

        return carry

    lax.fori_loop(0, N1, body, 0)

    for i in range(G1):
        c = w + NW * (G1 * (N1 - 1) + i)

        @pl.when(c < FULL_COLS)
        def _():
            pltpu.make_async_copy(
                obufs[i],
                tabRM.at[pl.ds(pl.multiple_of(c * 64, 8), 64)],
                osem).wait()

    @pl.when(w == 0)
    def _():
        pltpu.async_copy(
            tabT.at[:, pl.ds(FULL_COLS * 128, 64)], tib, tsem)
        pltpu.make_async_copy(
            tabT.at[:, pl.ds(FULL_COLS * 128, 64)], tib, tsem).wait()
        _transpose_block(tib, tob, row_id, 32)
        pltpu.async_copy(tob, tabRM.at[pl.ds(TAIL_ROW0, 32)], tsem)
        pltpu.make_async_copy(
            tob, tabRM.at[pl.ds(TAIL_ROW0, 32)], tsem).wait()


_detr_call = pl.kernel(
    _detr_body,
    out_type=jax.ShapeDtypeStruct((RM_ROWS, 128), jnp.float32),
    mesh=_MESH,
    scratch_types=[
        pltpu.VMEM((64, 128), jnp.float32),
        pltpu.VMEM((64, 128), jnp.float32),
        pltpu.VMEM((64, 128), jnp.float32),
        pltpu.VMEM((64, 128), jnp.float32),
        pltpu.VMEM((64, 128), jnp.float32),
        pltpu.VMEM((64, 128), jnp.float32),
        pltpu.VMEM((64, 128), jnp.float32),
        pltpu.VMEM((64, 128), jnp.float32),
        pltpu.VMEM((64, 64), jnp.float32),
        pltpu.VMEM((32, 128), jnp.float32),
        pltpu.SemaphoreType.DMA,
        pltpu.SemaphoreType.DMA,
        pltpu.SemaphoreType.DMA,
    ],
    compiler_params=_PARAMS,
)


def _gather_body(tokT, tabRM, outP, tok_v, idx0, idx1, par0, par1,
                 g0, g1, ov0, ov1, gsem0, gsem1, osem0, osem1):
    w = _wid()
    lane0 = pl.multiple_of(w * 128, 128)
    lanes = lax.iota(jnp.int32, LANES)
    row_id = [_splat(l * 16) + lanes for l in range(8)]
    idxs = (idx0, idx1)
    pars = (par0, par1)
    gbufs = (g0, g1)
    ovs = (ov0, ov1)
    gsems = (gsem0, gsem1)
    osems = (osem0, osem1)

    pltpu.sync_copy(tokT.at[:, pl.ds(lane0, 128)], tok_v)

    def prep_and_fire(p, b):
        for l in range(8):
            t = tok_v[p, pl.ds(l * 16, 16)]
            idxs[b][pl.ds(l * 16, 16)] = lax.shift_right_logical(t, 1)
            pars[b][pl.ds(l * 16, 16)] = lax.shift_left(
                jnp.bitwise_and(t, 1), 6)
        pltpu.async_copy(tabRM.at[idxs[b]], gbufs[b], gsems[b])

    def extract(p, b):
        par_l = [pars[b][pl.ds(l * 16, 16)] for l in range(8)]

        def ebody(e, carry):
            for l in range(8):
                v = plsc.load_gather(gbufs[b], [row_id[l], par_l[l] + e])
                ovs[b][e, pl.ds(l * 16, 16)] = v * SCALE
            return carry

        lax.fori_loop(0, EMB, ebody, 0)
        pltpu.async_copy(ovs[b], outP.at[p, :, pl.ds(lane0, 128)], osems[b])

    prep_and_fire(0, 0)

    def body(j, carry):
        for b in range(2):
            p = 2 * j + b

            @pl.when(p + 1 < SEQ)
            def _():
                prep_and_fire(p + 1, 1 - b)

            pltpu.make_async_copy(tabRM.at[idxs[b]], gbufs[b], gsems[b]).wait()

            @pl.when(p >= 2)
            def _():
                pltpu.make_async_copy(
                    ovs[b], outP.at[p, :, pl.ds(lane0, 128)], osems[b]).wait()

            extract(p, b)
        return carry

    lax.fori_loop(0, SEQ // 2, body, 0)

    for b in range(2):
        p = SEQ - 2 + b
        pltpu.make_async_copy(
            ovs[b], outP.at[p, :, pl.ds(lane0, 128)], osems[b]).wait()


_gather_call = pl.kernel(
    _gather_body,
    out_type=jax.ShapeDtypeStruct((SEQ, EMB, BATCH), jnp.float32),
    mesh=_MESH,
    scratch_types=[
        pltpu.VMEM((SEQ, 128), jnp.int32),
        pltpu.VMEM((128,), jnp.int32),
        pltpu.VMEM((128,), jnp.int32),
        pltpu.VMEM((128,), jnp.int32),
        pltpu.VMEM((128,), jnp.int32),
        pltpu.VMEM((128, 128), jnp.float32),
        pltpu.VMEM((128, 128), jnp.float32),
        pltpu.VMEM((EMB, 128), jnp.float32),
        pltpu.VMEM((EMB, 128), jnp.float32),
        pltpu.SemaphoreType.DMA,
        pltpu.SemaphoreType.DMA,
        pltpu.SemaphoreType.DMA,
        pltpu.SemaphoreType.DMA,
    ],
    compiler_params=_PARAMS,
)


@jax.jit
def kernel(tokens, table):
    tokT = tokens.astype(jnp.int32).T
    tabT = table.T
    tabRM = _detr_call(tabT)
    outP = _gather_call(tokT, tabRM)
    return outP.transpose(2, 0, 1)


# speedup vs baseline: 1.3762x; 1.3762x over previous
import math

import jax
import jax.numpy as jnp
from jax import lax
from jax.experimental import pallas as pl
from jax.experimental.pallas import tpu as pltpu
from jax.experimental.pallas import tpu_sc as plsc

VOCAB = 1000000
EMB = 64
BATCH = 4096
SEQ = 200
SCALE = math.sqrt(EMB)

NC = 2
NS = 16
NW = NC * NS
LANES = 16

TCOLS = (VOCAB + 127) // 128
FULL_COLS = VOCAB // 128
RM_ROWS = TCOLS * 64
TAIL_ROW0 = FULL_COLS * 64

G1 = 4
N1 = 62

_MESH = plsc.VectorSubcoreMesh(
    core_axis_name="c", subcore_axis_name="s", num_cores=NC, num_subcores=NS
)
_PARAMS = pltpu.CompilerParams(
    use_tc_tiling_on_sc=True, needs_layout_passes=False)


def _wid():
    return lax.axis_index("s") * NC + lax.axis_index("c")


def _splat(x):
    return jnp.zeros((LANES,), jnp.int32) + x


def _transpose_block(ibuf, obuf, row_id, n_q):
    def qbody(q2, carry):
        vs = []
        for dq in range(2):
            for half in range(2):
                col = _splat(2 * (2 * q2 + dq) + half)
                for l in range(4):
                    vs.append(plsc.load_gather(ibuf, [row_id[l], col]))
        i = 0
        for dq in range(2):
            for half in range(2):
                for l in range(4):
                    obuf[2 * q2 + dq, pl.ds(half * 64 + l * 16, 16)] = vs[i]
                    i += 1
        return carry

    lax.fori_loop(0, n_q // 2, qbody, 0)


def _detr_body(tabT, tabRM, i0, i1, i2, i3, o0, o1, o2, o3, tib, tob,
               isem, osem, tsem):
    w = _wid()
    lanes = lax.iota(jnp.int32, LANES)
    row_id = [_splat(l * 16) + lanes for l in range(4)]
    ibufs = (i0, i1, i2, i3)
    obufs = (o0, o1, o2, o3)

    def body(k, carry):
        cols = [w + NW * (G1 * k + i) for i in range(G1)]
        for i in range(G1):
            c = cols[i]

            @pl.when(c < FULL_COLS)
            def _():
                pltpu.async_copy(
                    tabT.at[:, pl.ds(pl.multiple_of(c * 128, 128), 128)],
                    ibufs[i], isem)

        for i in range(G1):
            c = cols[i]
            cprev = c - NW * G1

            @pl.when(c < FULL_COLS)
            def _():
                pltpu.make_async_copy(
                    tabT.at[:, pl.ds(pl.multiple_of(c * 128, 128), 128)],
                    ibufs[i], isem).wait()

            @pl.when(cprev >= 0)
            def _():
                pltpu.make_async_copy(
                    obufs[i],
                    tabRM.at[pl.ds(pl.multiple_of(cprev * 64, 8), 64)],
                    osem).wait()

            @pl.when(c < FULL_COLS)
            def _():
                _transpose_block(ibufs[i], obufs[i], row_id, 64)
                pltpu.async_copy(
                    obufs[i],
                    tabRM.at[pl.ds(pl.multiple_of(c * 64, 8), 64)],
                    osem)

        return carry

    lax.fori_loop(0, N1, body, 0)

    for i in range(G1):
        c = w + NW * (G1 * (N1 - 1) + i)

        @pl.when(c < FULL_COLS)
        def _():
            pltpu.make_async_copy(
                obufs[i],
                tabRM.at[pl.ds(pl.multiple_of(c * 64, 8), 64)],
                osem).wait()

    @pl.when(w == 0)
    def _():
        pltpu.async_copy(
            tabT.at[:, pl.ds(FULL_COLS * 128, 64)], tib, tsem)
        pltpu.make_async_copy(
            tabT.at[:, pl.ds(FULL_COLS * 128, 64)], tib, tsem).wait()
        _transpose_block(tib, tob, row_id, 32)
        pltpu.async_copy(tob, tabRM.at[pl.ds(TAIL_ROW0, 32)], tsem)
        pltpu.make_async_copy(
            tob, tabRM.at[pl.ds(TAIL_ROW0, 32)], tsem).wait()


_detr_call = pl.kernel(
    _detr_body,
    out_type=jax.ShapeDtypeStruct((RM_ROWS, 128), jnp.float32),
    mesh=_MESH,
    scratch_types=[
        pltpu.VMEM((64, 128), jnp.float32),
        pltpu.VMEM((64, 128), jnp.float32),
        pltpu.VMEM((64, 128), jnp.float32),
        pltpu.VMEM((64, 128), jnp.float32),
        pltpu.VMEM((64, 128), jnp.float32),
        pltpu.VMEM((64, 128), jnp.float32),
        pltpu.VMEM((64, 128), jnp.float32),
        pltpu.VMEM((64, 128), jnp.float32),
        pltpu.VMEM((64, 64), jnp.float32),
        pltpu.VMEM((32, 128), jnp.float32),
        pltpu.SemaphoreType.DMA,
        pltpu.SemaphoreType.DMA,
        pltpu.SemaphoreType.DMA,
    ],
    compiler_params=_PARAMS,
)


def _gather_body(tokT, tabRM, outP, tok_v, idx0, idx1, par0, par1,
                 g0, g1, ov0, ov1, gsem0, gsem1, osem0, osem1):
    w = _wid()
    lane0 = pl.multiple_of(w * 128, 128)
    lanes = lax.iota(jnp.int32, LANES)
    row_id = [_splat(l * 16) + lanes for l in range(8)]
    idxs = (idx0, idx1)
    pars = (par0, par1)
    gbufs = (g0, g1)
    ovs = (ov0, ov1)
    gsems = (gsem0, gsem1)
    osems = (osem0, osem1)

    pltpu.sync_copy(tokT.at[:, pl.ds(lane0, 128)], tok_v)

    def prep_and_fire(p, b):
        for l in range(8):
            t = tok_v[p, pl.ds(l * 16, 16)]
            idxs[b][pl.ds(l * 16, 16)] = lax.shift_right_logical(t, 1)
            pars[b][pl.ds(l * 16, 16)] = lax.shift_left(
                jnp.bitwise_and(t, 1), 6)
        pltpu.async_copy(tabRM.at[idxs[b]], gbufs[b], gsems[b])

    def extract(p, b):
        par_l = [pars[b][pl.ds(l * 16, 16)] for l in range(8)]

        def ebody(e2, carry):
            vs = []
            for de in range(2):
                e = 2 * e2 + de
                for l in range(8):
                    vs.append(
                        plsc.load_gather(gbufs[b], [row_id[l], par_l[l] + e]))
            i = 0
            for de in range(2):
                for l in range(8):
                    ovs[b][2 * e2 + de, pl.ds(l * 16, 16)] = vs[i] * SCALE
                    i += 1
            return carry

        lax.fori_loop(0, EMB // 2, ebody, 0)
        pltpu.async_copy(ovs[b], outP.at[p, :, pl.ds(lane0, 128)], osems[b])

    prep_and_fire(0, 0)

    def body(j, carry):
        for b in range(2):
            p = 2 * j + b

            @pl.when(p + 1 < SEQ)
            def _():
                prep_and_fire(p + 1, 1 - b)

            pltpu.make_async_copy(tabRM.at[idxs[b]], gbufs[b], gsems[b]).wait()

            @pl.when(p >= 2)
            def _():
                pltpu.make_async_copy(
                    ovs[b], outP.at[p, :, pl.ds(lane0, 128)], osems[b]).wait()

            extract(p, b)
        return carry

    lax.fori_loop(0, SEQ // 2, body, 0)

    for b in range(2):
        p = SEQ - 2 + b
        pltpu.make_async_copy(
            ovs[b], outP.at[p, :, pl.ds(lane0, 128)], osems[b]).wait()


_gather_call = pl.kernel(
    _gather_body,
    out_type=jax.ShapeDtypeStruct((SEQ, EMB, BATCH), jnp.float32),
    mesh=_MESH,
    scratch_types=[
        pltpu.VMEM((SEQ, 128), jnp.int32),
        pltpu.VMEM((128,), jnp.int32),
        pltpu.VMEM((128,), jnp.int32),
        pltpu.VMEM((128,), jnp.int32),
        pltpu.VMEM((128,), jnp.int32),
        pltpu.VMEM((128, 128), jnp.float32),
        pltpu.VMEM((128, 128), jnp.float32),
        pltpu.VMEM((EMB, 128), jnp.float32),
        pltpu.VMEM((EMB, 128), jnp.float32),
        pltpu.SemaphoreType.DMA,
        pltpu.SemaphoreType.DMA,
        pltpu.SemaphoreType.DMA,
        pltpu.SemaphoreType.DMA,
    ],
    compiler_params=_PARAMS,
)


@jax.jit
def kernel(tokens, table):
    tokT = tokens.astype(jnp.int32).T
    tabT = table.T
    tabRM = _detr_call(tabT)
    outP = _gather_call(tokT, tabRM)
    return outP.transpose(2, 0, 1)


# speedup vs baseline: 1.4211x; 1.0327x over previous
import math

import jax
import jax.numpy as jnp
from jax import lax
from jax.experimental import pallas as pl
from jax.experimental.pallas import tpu as pltpu
from jax.experimental.pallas import tpu_sc as plsc

VOCAB = 1000000
EMB = 64
BATCH = 4096
SEQ = 200
SCALE = math.sqrt(EMB)

NC = 2
NS = 16
NW = NC * NS
LANES = 16

TCOLS = (VOCAB + 127) // 128
FULL_COLS = VOCAB // 128
RM_ROWS = TCOLS * 64
TAIL_ROW0 = FULL_COLS * 64

SCHUNKS = FULL_COLS // 2
N1 = 62

_MESH = plsc.VectorSubcoreMesh(
    core_axis_name="c", subcore_axis_name="s", num_cores=NC, num_subcores=NS
)
_PARAMS = pltpu.CompilerParams(
    use_tc_tiling_on_sc=True, needs_layout_passes=False)


def _wid():
    return lax.axis_index("s") * NC + lax.axis_index("c")


def _splat(x):
    return jnp.zeros((LANES,), jnp.int32) + x


def _transpose_block(ibuf, obuf, row_id, n_q):
    def qbody(q2, carry):
        vs = []
        for dq in range(2):
            for half in range(2):
                col = _splat(2 * (2 * q2 + dq) + half)
                for l in range(4):
                    vs.append(plsc.load_gather(ibuf, [row_id[l], col]))
        i = 0
        for dq in range(2):
            for half in range(2):
                for l in range(4):
                    obuf[2 * q2 + dq, pl.ds(half * 64 + l * 16, 16)] = vs[i]
                    i += 1
        return carry

    lax.fori_loop(0, n_q // 2, qbody, 0)


def _detr_body(tabT, tabRM, i0, i1, o0, o1, tib, tob, isem, osem, tsem):
    w = _wid()
    lanes = lax.iota(jnp.int32, LANES)
    row_id = [_splat(l * 16) + lanes for l in range(4)]
    ibufs = (i0, i1)
    obufs = (o0, o1)

    def fire_in(sc, b):
        @pl.when(sc < SCHUNKS)
        def _():
            for i in range(8):
                pltpu.async_copy(
                    tabT.at[pl.ds(8 * i, 8),
                            pl.ds(pl.multiple_of(sc * 256, 128), 256)],
                    ibufs[b].at[pl.ds(8 * i, 8), :],
                    isem)

    def wait_in(sc, b):
        @pl.when(sc < SCHUNKS)
        def _():
            for i in range(8):
                pltpu.make_async_copy(
                    tabT.at[pl.ds(8 * i, 8),
                            pl.ds(pl.multiple_of(sc * 256, 128), 256)],
                    ibufs[b].at[pl.ds(8 * i, 8), :],
                    isem).wait()

    def out_dma(sc, b):
        return pltpu.make_async_copy(
            obufs[b], tabRM.at[pl.ds(pl.multiple_of(sc * 128, 8), 128)],
            osem)

    fire_in(w, 0)

    def body(k2, carry):
        for b in range(2):
            k = 2 * k2 + b
            sc = w + NW * k
            fire_in(sc + NW, 1 - b)
            wait_in(sc, b)

            @pl.when(sc - 2 * NW >= 0)
            def _():
                out_dma(sc - 2 * NW, b).wait()

            @pl.when(sc < SCHUNKS)
            def _():
                _transpose_block(ibufs[b], obufs[b], row_id, 128)
                out_dma(sc, b).start()

        return carry

    lax.fori_loop(0, N1, body, 0)

    for b in range(2):
        sc = w + NW * (2 * (N1 - 1) + b)

        @pl.when(sc < SCHUNKS)
        def _():
            out_dma(sc, b).wait()

    @pl.when(w == 0)
    def _():
        pltpu.async_copy(
            tabT.at[:, pl.ds(FULL_COLS * 128, 64)], tib, tsem)
        pltpu.make_async_copy(
            tabT.at[:, pl.ds(FULL_COLS * 128, 64)], tib, tsem).wait()
        _transpose_block(tib, tob, row_id, 32)
        pltpu.async_copy(tob, tabRM.at[pl.ds(TAIL_ROW0, 32)], tsem)
        pltpu.make_async_copy(
            tob, tabRM.at[pl.ds(TAIL_ROW0, 32)], tsem).wait()


_detr_call = pl.kernel(
    _detr_body,
    out_type=jax.ShapeDtypeStruct((RM_ROWS, 128), jnp.float32),
    mesh=_MESH,
    scratch_types=[
        pltpu.VMEM((64, 256), jnp.float32),
        pltpu.VMEM((64, 256), jnp.float32),
        pltpu.VMEM((128, 128), jnp.float32),
        pltpu.VMEM((128, 128), jnp.float32),
        pltpu.VMEM((64, 64), jnp.float32),
        pltpu.VMEM((32, 128), jnp.float32),
        pltpu.SemaphoreType.DMA,
        pltpu.SemaphoreType.DMA,
        pltpu.SemaphoreType.DMA,
    ],
    compiler_params=_PARAMS,
)


def _gather_body(tokT, tabRM, outP, tok_v, idx0, idx1, par0, par1,
                 g0, g1, ov0, ov1, gsem0, gsem1, osem0, osem1):
    w = _wid()
    lane0 = pl.multiple_of(w * 128, 128)
    lanes = lax.iota(jnp.int32, LANES)
    row_id = [_splat(l * 16) + lanes for l in range(8)]
    idxs = (idx0, idx1)
    pars = (par0, par1)
    gbufs = (g0, g1)
    ovs = (ov0, ov1)
    gsems = (gsem0, gsem1)
    osems = (osem0, osem1)

    pltpu.sync_copy(tokT.at[:, pl.ds(lane0, 128)], tok_v)

    def prep_and_fire(p, b):
        for l in range(8):
            t = tok_v[p, pl.ds(l * 16, 16)]
            idxs[b][pl.ds(l * 16, 16)] = lax.shift_right_logical(t, 1)
            pars[b][pl.ds(l * 16, 16)] = lax.shift_left(
                jnp.bitwise_and(t, 1), 6)
        for l in range(8):
            pltpu.async_copy(
                tabRM.at[idxs[b].at[pl.ds(l * 16, 16)]],
                gbufs[b].at[pl.ds(l * 16, 16)],
                gsems[b])

    def wait_gather(b):
        for l in range(8):
            pltpu.make_async_copy(
                tabRM.at[idxs[b].at[pl.ds(l * 16, 16)]],
                gbufs[b].at[pl.ds(l * 16, 16)],
                gsems[b]).wait()

    def extract(p, b):
        par_l = [pars[b][pl.ds(l * 16, 16)] for l in range(8)]

        def ebody(e2, carry):
            vs = []
            for de in range(2):
                e = 2 * e2 + de
                for l in range(8):
                    vs.append(
                        plsc.load_gather(gbufs[b], [row_id[l], par_l[l] + e]))
            i = 0
            for de in range(2):
                for l in range(8):
                    ovs[b][2 * e2 + de, pl.ds(l * 16, 16)] = vs[i] * SCALE
                    i += 1
            return carry

        lax.fori_loop(0, EMB // 2, ebody, 0)
        pltpu.async_copy(ovs[b], outP.at[p, :, pl.ds(lane0, 128)], osems[b])

    prep_and_fire(0, 0)

    def body(j, carry):
        for b in range(2):
            p = 2 * j + b

            @pl.when(p + 1 < SEQ)
            def _():
                prep_and_fire(p + 1, 1 - b)

            wait_gather(b)

            @pl.when(p >= 2)
            def _():
                pltpu.make_async_copy(
                    ovs[b], outP.at[p, :, pl.ds(lane0, 128)], osems[b]).wait()

            extract(p, b)
        return carry

    lax.fori_loop(0, SEQ // 2, body, 0)

    for b in range(2):
        p = SEQ - 2 + b
        pltpu.make_async_copy(
            ovs[b], outP.at[p, :, pl.ds(lane0, 128)], osems[b]).wait()


_gather_call = pl.kernel(
    _gather_body,
    out_type=jax.ShapeDtypeStruct((SEQ, EMB, BATCH), jnp.float32),
    mesh=_MESH,
    scratch_types=[
        pltpu.VMEM((SEQ, 128), jnp.int32),
        pltpu.VMEM((128,), jnp.int32),
        pltpu.VMEM((128,), jnp.int32),
        pltpu.VMEM((128,), jnp.int32),
        pltpu.VMEM((128,), jnp.int32),
        pltpu.VMEM((128, 128), jnp.float32),
        pltpu.VMEM((128, 128), jnp.float32),
        pltpu.VMEM((EMB, 128), jnp.float32),
        pltpu.VMEM((EMB, 128), jnp.float32),
        pltpu.SemaphoreType.DMA,
        pltpu.SemaphoreType.DMA,
        pltpu.SemaphoreType.DMA,
        pltpu.SemaphoreType.DMA,
    ],
    compiler_params=_PARAMS,
)


@jax.jit
def kernel(tokens, table):
    tokT = tokens.astype(jnp.int32).T
    tabT = table.T
    tabRM = _detr_call(tabT)
    outP = _gather_call(tokT, tabRM)
    return outP.transpose(2, 0, 1)


# speedup vs baseline: 2.0736x; 1.4591x over previous
import math

import jax
import jax.numpy as jnp
from jax import lax
from jax.experimental import pallas as pl
from jax.experimental.pallas import tpu as pltpu
from jax.experimental.pallas import tpu_sc as plsc

VOCAB = 1000000
EMB = 64
BATCH = 4096
SEQ = 200
SCALE = math.sqrt(EMB)

NC = 2
NS = 16
NW = NC * NS
LANES = 16

TRW = 2048
TRG = (VOCAB + TRW - 1) // TRW
RM_ROWS = TRG * (TRW // 2)

_MESH = plsc.VectorSubcoreMesh(
    core_axis_name="c", subcore_axis_name="s", num_cores=NC, num_subcores=NS
)
_PARAMS = pltpu.CompilerParams(
    use_tc_tiling_on_sc=True, needs_layout_passes=False)


def _tr_body(in_ref, out_ref):
    x = in_ref[...]
    out_ref[:, 0:64] = x[:, 0:1024].T
    out_ref[:, 64:128] = x[:, 1024:2048].T


_tr_call = pl.pallas_call(
    _tr_body,
    grid=(TRG,),
    in_specs=[pl.BlockSpec((EMB, TRW), lambda i: (0, i))],
    out_specs=pl.BlockSpec((TRW // 2, 128), lambda i: (i, 0)),
    out_shape=jax.ShapeDtypeStruct((RM_ROWS, 128), jnp.float32),
)


def _wid():
    return lax.axis_index("s") * NC + lax.axis_index("c")


def _splat(x):
    return jnp.zeros((LANES,), jnp.int32) + x


def _gather_body(tokT, tabRM, outP, tok_v, idx0, idx1, par0, par1,
                 g0, g1, ov0, ov1, gsem0, gsem1, osem0, osem1):
    w = _wid()
    lane0 = pl.multiple_of(w * 128, 128)
    lanes = lax.iota(jnp.int32, LANES)
    row_id = [_splat(l * 16) + lanes for l in range(8)]
    idxs = (idx0, idx1)
    pars = (par0, par1)
    gbufs = (g0, g1)
    ovs = (ov0, ov1)
    gsems = (gsem0, gsem1)
    osems = (osem0, osem1)

    pltpu.sync_copy(tokT.at[:, pl.ds(lane0, 128)], tok_v)

    def prep_and_fire(p, b):
        for l in range(8):
            t = tok_v[p, pl.ds(l * 16, 16)]
            idxs[b][pl.ds(l * 16, 16)] = jnp.bitwise_or(
                jnp.bitwise_and(lax.shift_right_logical(t, 1), -1024),
                jnp.bitwise_and(t, 1023))
            pars[b][pl.ds(l * 16, 16)] = lax.shift_right_logical(
                jnp.bitwise_and(t, 1024), 4)
        for l in range(8):
            pltpu.async_copy(
                tabRM.at[idxs[b].at[pl.ds(l * 16, 16)]],
                gbufs[b].at[pl.ds(l * 16, 16)],
                gsems[b])

    def wait_gather(b):
        for l in range(8):
            pltpu.make_async_copy(
                tabRM.at[idxs[b].at[pl.ds(l * 16, 16)]],
                gbufs[b].at[pl.ds(l * 16, 16)],
                gsems[b]).wait()

    def extract(p, b):
        par_l = [pars[b][pl.ds(l * 16, 16)] for l in range(8)]

        def ebody(e2, carry):
            vs = []
            for de in range(2):
                e = 2 * e2 + de
                for l in range(8):
                    vs.append(
                        plsc.load_gather(gbufs[b], [row_id[l], par_l[l] + e]))
            i = 0
            for de in range(2):
                for l in range(8):
                    ovs[b][2 * e2 + de, pl.ds(l * 16, 16)] = vs[i] * SCALE
                    i += 1
            return carry

        lax.fori_loop(0, EMB // 2, ebody, 0)
        pltpu.async_copy(ovs[b], outP.at[p, :, pl.ds(lane0, 128)], osems[b])

    prep_and_fire(0, 0)

    def body(j, carry):
        for b in range(2):
            p = 2 * j + b

            @pl.when(p + 1 < SEQ)
            def _():
                prep_and_fire(p + 1, 1 - b)

            wait_gather(b)

            @pl.when(p >= 2)
            def _():
                pltpu.make_async_copy(
                    ovs[b], outP.at[p, :, pl.ds(lane0, 128)], osems[b]).wait()

            extract(p, b)
        return carry

    lax.fori_loop(0, SEQ // 2, body, 0)

    for b in range(2):
        p = SEQ - 2 + b
        pltpu.make_async_copy(
            ovs[b], outP.at[p, :, pl.ds(lane0, 128)], osems[b]).wait()


_gather_call = pl.kernel(
    _gather_body,
    out_type=jax.ShapeDtypeStruct((SEQ, EMB, BATCH), jnp.float32),
    mesh=_MESH,
    scratch_types=[
        pltpu.VMEM((SEQ, 128), jnp.int32),
        pltpu.VMEM((128,), jnp.int32),
        pltpu.VMEM((128,), jnp.int32),
        pltpu.VMEM((128,), jnp.int32),
        pltpu.VMEM((128,), jnp.int32),
        pltpu.VMEM((128, 128), jnp.float32),
        pltpu.VMEM((128, 128), jnp.float32),
        pltpu.VMEM((EMB, 128), jnp.float32),
        pltpu.VMEM((EMB, 128), jnp.float32),
        pltpu.SemaphoreType.DMA,
        pltpu.SemaphoreType.DMA,
        pltpu.SemaphoreType.DMA,
        pltpu.SemaphoreType.DMA,
    ],
    compiler_params=_PARAMS,
)


@jax.jit
def kernel(tokens, table):
    tokT = tokens.astype(jnp.int32).T
    tabT = table.T
    tabRM = _tr_call(tabT)
    outP = _gather_call(tokT, tabRM)
    return outP.transpose(2, 0, 1)


# speedup vs baseline: 2.2677x; 1.0936x over previous
import math

import jax
import jax.numpy as jnp
from jax import lax
from jax.experimental import pallas as pl
from jax.experimental.pallas import tpu as pltpu
from jax.experimental.pallas import tpu_sc as plsc

VOCAB = 1000000
EMB = 64
BATCH = 4096
SEQ = 200
SCALE = math.sqrt(EMB)

NC = 2
NS = 16
NW = NC * NS
LANES = 16

TRW = 4096
TRG = (VOCAB + TRW - 1) // TRW
RM_ROWS = TRG * (TRW // 2)

_MESH = plsc.VectorSubcoreMesh(
    core_axis_name="c", subcore_axis_name="s", num_cores=NC, num_subcores=NS
)
_PARAMS = pltpu.CompilerParams(
    use_tc_tiling_on_sc=True, needs_layout_passes=False)


def _tr_body(in_ref, out_ref):
    x = in_ref[...]
    out_ref[:, 0:64] = x[:, 0:2048].T
    out_ref[:, 64:128] = x[:, 2048:4096].T


_tr_call = pl.pallas_call(
    _tr_body,
    grid=(TRG,),
    in_specs=[pl.BlockSpec((EMB, TRW), lambda i: (0, i))],
    out_specs=pl.BlockSpec((TRW // 2, 128), lambda i: (i, 0)),
    out_shape=jax.ShapeDtypeStruct((RM_ROWS, 128), jnp.float32),
)


def _wid():
    return lax.axis_index("s") * NC + lax.axis_index("c")


def _splat(x):
    return jnp.zeros((LANES,), jnp.int32) + x


def _gather_body(tokT, tabRM, outP, tok_v, idx0, idx1, par0, par1,
                 g0, g1, ov0, ov1, gsem0, gsem1, osem0, osem1):
    w = _wid()
    lane0 = pl.multiple_of(w * 128, 128)
    lanes = lax.iota(jnp.int32, LANES)
    row_id = [_splat(l * 16) + lanes for l in range(8)]
    idxs = (idx0, idx1)
    pars = (par0, par1)
    gbufs = (g0, g1)
    ovs = (ov0, ov1)
    gsems = (gsem0, gsem1)
    osems = (osem0, osem1)

    pltpu.sync_copy(tokT.at[:, pl.ds(lane0, 128)], tok_v)

    def prep_and_fire(p, b):
        for l in range(8):
            t = tok_v[p, pl.ds(l * 16, 16)]
            idxs[b][pl.ds(l * 16, 16)] = jnp.bitwise_or(
                jnp.bitwise_and(lax.shift_right_logical(t, 1), -2048),
                jnp.bitwise_and(t, 2047))
            pars[b][pl.ds(l * 16, 16)] = lax.shift_right_logical(
                jnp.bitwise_and(t, 2048), 5)
        for l in range(8):
            pltpu.async_copy(
                tabRM.at[idxs[b].at[pl.ds(l * 16, 16)]],
                gbufs[b].at[pl.ds(l * 16, 16)],
                gsems[b])

    def wait_gather(b):
        for l in range(8):
            pltpu.make_async_copy(
                tabRM.at[idxs[b].at[pl.ds(l * 16, 16)]],
                gbufs[b].at[pl.ds(l * 16, 16)],
                gsems[b]).wait()

    def extract(p, b):
        par_l = [pars[b][pl.ds(l * 16, 16)] for l in range(8)]

        def ebody(e2, carry):
            vs = []
            for de in range(2):
                e = 2 * e2 + de
                for l in range(8):
                    vs.append(
                        plsc.load_gather(gbufs[b], [row_id[l], par_l[l] + e]))
            i = 0
            for de in range(2):
                for l in range(8):
                    ovs[b][2 * e2 + de, pl.ds(l * 16, 16)] = vs[i] * SCALE
                    i += 1
            return carry

        lax.fori_loop(0, EMB // 2, ebody, 0)
        pltpu.async_copy(ovs[b], outP.at[p, :, pl.ds(lane0, 128)], osems[b])

    prep_and_fire(0, 0)

    def body(j, carry):
        for b in range(2):
            p = 2 * j + b

            @pl.when(p + 1 < SEQ)
            def _():
                prep_and_fire(p + 1, 1 - b)

            wait_gather(b)

            @pl.when(p >= 2)
            def _():
                pltpu.make_async_copy(
                    ovs[b], outP.at[p, :, pl.ds(lane0, 128)], osems[b]).wait()

            extract(p, b)
        return carry

    lax.fori_loop(0, SEQ // 2, body, 0)

    for b in range(2):
        p = SEQ - 2 + b
        pltpu.make_async_copy(
            ovs[b], outP.at[p, :, pl.ds(lane0, 128)], osems[b]).wait()


_gather_call = pl.kernel(
    _gather_body,
    out_type=jax.ShapeDtypeStruct((SEQ, EMB, BATCH), jnp.float32),
    mesh=_MESH,
    scratch_types=[
        pltpu.VMEM((SEQ, 128), jnp.int32),
        pltpu.VMEM((128,), jnp.int32),
        pltpu.VMEM((128,), jnp.int32),
        pltpu.VMEM((128,), jnp.int32),
        pltpu.VMEM((128,), jnp.int32),
        pltpu.VMEM((128, 128), jnp.float32),
        pltpu.VMEM((128, 128), jnp.float32),
        pltpu.VMEM((EMB, 128), jnp.float32),
        pltpu.VMEM((EMB, 128), jnp.float32),
        pltpu.SemaphoreType.DMA,
        pltpu.SemaphoreType.DMA,
        pltpu.SemaphoreType.DMA,
        pltpu.SemaphoreType.DMA,
    ],
    compiler_params=_PARAMS,
)


@jax.jit
def kernel(tokens, table):
    tokT = tokens.astype(jnp.int32).T
    tabT = table.T
    tabRM = _tr_call(tabT)
    outP = _gather_call(tokT, tabRM)
    return outP.transpose(2, 0, 1)


# speedup vs baseline: 2.3916x; 1.0546x over previous
import math

import jax
import jax.numpy as jnp
from jax import lax
from jax.experimental import pallas as pl
from jax.experimental.pallas import tpu as pltpu
from jax.experimental.pallas import tpu_sc as plsc

VOCAB = 1000000
EMB = 64
BATCH = 4096
SEQ = 200
SCALE = math.sqrt(EMB)

NC = 2
NS = 16
NW = NC * NS
LANES = 16

TRW = 8192
TRG = (VOCAB + TRW - 1) // TRW
RM_ROWS = TRG * (TRW // 2)

_MESH = plsc.VectorSubcoreMesh(
    core_axis_name="c", subcore_axis_name="s", num_cores=NC, num_subcores=NS
)
_PARAMS = pltpu.CompilerParams(
    use_tc_tiling_on_sc=True, needs_layout_passes=False)


def _tr_body(in_ref, out_ref):
    x = in_ref[...]
    out_ref[:, 0:64] = x[:, 0:4096].T
    out_ref[:, 64:128] = x[:, 4096:8192].T


_tr_call = pl.pallas_call(
    _tr_body,
    grid=(TRG,),
    in_specs=[pl.BlockSpec((EMB, TRW), lambda i: (0, i))],
    out_specs=pl.BlockSpec((TRW // 2, 128), lambda i: (i, 0)),
    out_shape=jax.ShapeDtypeStruct((RM_ROWS, 128), jnp.float32),
)


def _wid():
    return lax.axis_index("s") * NC + lax.axis_index("c")


def _splat(x):
    return jnp.zeros((LANES,), jnp.int32) + x


def _gather_body(tokT, tabRM, outP, tok_v, idx0, idx1, par0, par1,
                 g0, g1, ov0, ov1, gsem0, gsem1, osem0, osem1):
    w = _wid()
    lane0 = pl.multiple_of(w * 128, 128)
    lanes = lax.iota(jnp.int32, LANES)
    row_id = [_splat(l * 16) + lanes for l in range(8)]
    idxs = (idx0, idx1)
    pars = (par0, par1)
    gbufs = (g0, g1)
    ovs = (ov0, ov1)
    gsems = (gsem0, gsem1)
    osems = (osem0, osem1)

    pltpu.sync_copy(tokT.at[:, pl.ds(lane0, 128)], tok_v)

    def prep_and_fire(p, b):
        for l in range(8):
            t = tok_v[p, pl.ds(l * 16, 16)]
            idxs[b][pl.ds(l * 16, 16)] = jnp.bitwise_or(
                jnp.bitwise_and(lax.shift_right_logical(t, 1), -4096),
                jnp.bitwise_and(t, 4095))
            pars[b][pl.ds(l * 16, 16)] = lax.shift_right_logical(
                jnp.bitwise_and(t, 4096), 6)
        for l in range(8):
            pltpu.async_copy(
                tabRM.at[idxs[b].at[pl.ds(l * 16, 16)]],
                gbufs[b].at[pl.ds(l * 16, 16)],
                gsems[b])

    def wait_gather(b):
        for l in range(8):
            pltpu.make_async_copy(
                tabRM.at[idxs[b].at[pl.ds(l * 16, 16)]],
                gbufs[b].at[pl.ds(l * 16, 16)],
                gsems[b]).wait()

    def extract(p, b):
        par_l = [pars[b][pl.ds(l * 16, 16)] for l in range(8)]

        def ebody(e2, carry):
            vs = []
            for de in range(2):
                e = 2 * e2 + de
                for l in range(8):
                    vs.append(
                        plsc.load_gather(gbufs[b], [row_id[l], par_l[l] + e]))
            i = 0
            for de in range(2):
                for l in range(8):
                    ovs[b][2 * e2 + de, pl.ds(l * 16, 16)] = vs[i] * SCALE
                    i += 1
            return carry

        lax.fori_loop(0, EMB // 2, ebody, 0)
        pltpu.async_copy(ovs[b], outP.at[p, :, pl.ds(lane0, 128)], osems[b])

    prep_and_fire(0, 0)

    def body(j, carry):
        for b in range(2):
            p = 2 * j + b

            @pl.when(p + 1 < SEQ)
            def _():
                prep_and_fire(p + 1, 1 - b)

            wait_gather(b)

            @pl.when(p >= 2)
            def _():
                pltpu.make_async_copy(
                    ovs[b], outP.at[p, :, pl.ds(lane0, 128)], osems[b]).wait()

            extract(p, b)
        return carry

    lax.fori_loop(0, SEQ // 2, body, 0)

    for b in range(2):
        p = SEQ - 2 + b
        pltpu.make_async_copy(
            ovs[b], outP.at[p, :, pl.ds(lane0, 128)], osems[b]).wait()


_gather_call = pl.kernel(
    _gather_body,
    out_type=jax.ShapeDtypeStruct((SEQ, EMB, BATCH), jnp.float32),
    mesh=_MESH,
    scratch_types=[
        pltpu.VMEM((SEQ, 128), jnp.int32),
        pltpu.VMEM((128,), jnp.int32),
        pltpu.VMEM((128,), jnp.int32),
        pltpu.VMEM((128,), jnp.int32),
        pltpu.VMEM((128,), jnp.int32),
        pltpu.VMEM((128, 128), jnp.float32),
        pltpu.VMEM((128, 128), jnp.float32),
        pltpu.VMEM((EMB, 128), jnp.float32),
        pltpu.VMEM((EMB, 128), jnp.float32),
        pltpu.SemaphoreType.DMA,
        pltpu.SemaphoreType.DMA,
        pltpu.SemaphoreType.DMA,
        pltpu.SemaphoreType.DMA,
    ],
    compiler_params=_PARAMS,
)


@jax.jit
def kernel(tokens, table):
    tokT = tokens.astype(jnp.int32).T
    tabT = table.T
    tabRM = _tr_call(tabT)
    outP = _gather_call(tokT, tabRM)
    return outP.transpose(2, 0, 1)
